# Initial kernel scaffold; baseline (speedup 1.0000x reference)
import jax
import jax.numpy as jnp
from jax.experimental import pallas as pl


def kernel(inputs, tables, W1, b1, W2, b2, W3, b3, Wo, bo, w_wide, w0):
    raise NotImplementedError("write your pallas kernel here")


# speedup vs baseline: 17.3467x; 17.3467x over previous
import functools

import jax
import jax.numpy as jnp
from jax.experimental import pallas as pl
from jax.experimental.layout import Layout, with_layout_constraint
from jax.experimental.pallas import tpu as pltpu
from jax.experimental.pallas import tpu_sc as plsc

V = 100000
D = 32
NF = 26
N_DENSE = 13
N_ONEHOT = 100
N_IN = N_DENSE + NF + N_ONEHOT

_BB = 1024


_NW = 32
_CH = 256


def _sc_gather(tables_flat, flat_idx, num_idx):
    b_per_w = num_idx // _NW
    n_ch = b_per_w // _CH
    mesh = plsc.VectorSubcoreMesh(core_axis_name="core", subcore_axis_name="subcore")

    @functools.partial(
        pl.kernel,
        out_type=jax.ShapeDtypeStruct((num_idx, D), jnp.float32),
        mesh=mesh,
        scratch_types=[
            pltpu.VMEM((b_per_w,), jnp.int32),
            pltpu.VMEM((2, _CH, D), jnp.float32),
            pltpu.SemaphoreType.DMA,
            pltpu.SemaphoreType.DMA,
        ],
    )
    def gather_kernel(tab_hbm, idx_hbm, out_hbm, idx_v, rows_v, sem0, sem1):
        wid = jax.lax.axis_index("subcore") * 2 + jax.lax.axis_index("core")
        base = wid * b_per_w
        pltpu.sync_copy(idx_hbm.at[pl.ds(base, b_per_w)], idx_v)

        pltpu.async_copy(
            tab_hbm.at[idx_v.at[pl.ds(0, _CH)]], rows_v.at[0], sem0)

        @pl.loop(0, n_ch, step=2)
        def _(c):
            off = c * _CH

            @pl.when(c + 1 < n_ch)
            def _():
                pltpu.async_copy(
                    tab_hbm.at[idx_v.at[pl.ds(off + _CH, _CH)]],
                    rows_v.at[1], sem1)

            pltpu.make_async_copy(
                tab_hbm.at[idx_v.at[pl.ds(off, _CH)]], rows_v.at[0],
                sem0).wait()
            pltpu.sync_copy(rows_v.at[0], out_hbm.at[pl.ds(base + off, _CH)])

            @pl.when(c + 1 < n_ch)
            def _():
                @pl.when(c + 2 < n_ch)
                def _():
                    pltpu.async_copy(
                        tab_hbm.at[idx_v.at[pl.ds(off + 2 * _CH, _CH)]],
                        rows_v.at[0], sem0)

                pltpu.make_async_copy(
                    tab_hbm.at[idx_v.at[pl.ds(off + _CH, _CH)]],
                    rows_v.at[1], sem1).wait()
                pltpu.sync_copy(
                    rows_v.at[1], out_hbm.at[pl.ds(base + off + _CH, _CH)])

    return gather_kernel(tables_flat, flat_idx)


def _mlp_body(inp_ref, emb_ref, w1_ref, b1_ref, w2_ref, b2_ref, w3_ref,
              b3_ref, wo_ref, wpad_ref, bias_ref, out_ref):
    x = emb_ref[...]
    h = jnp.maximum(jnp.dot(x, w1_ref[...], preferred_element_type=jnp.float32)
                    + b1_ref[...], 0.0)
    h = jnp.maximum(jnp.dot(h, w2_ref[...], preferred_element_type=jnp.float32)
                    + b2_ref[...], 0.0)
    h = jnp.maximum(jnp.dot(h, w3_ref[...], preferred_element_type=jnp.float32)
                    + b3_ref[...], 0.0)
    deep = jnp.dot(h, wo_ref[...], preferred_element_type=jnp.float32)
    wide = jnp.dot(inp_ref[...], wpad_ref[...], preferred_element_type=jnp.float32)
    z = 0.5 * (wide + deep + bias_ref[...])
    out_ref[...] = jax.nn.sigmoid(z)


def kernel(inputs, tables, W1, b1, W2, b2, W3, b3, Wo, bo, w_wide, w0):
    b = inputs.shape[0]
    num_idx = b * NF
    idx = jax.lax.stop_gradient(inputs[:, N_DENSE:N_DENSE + NF]).astype(jnp.int32)
    flat_idx = (idx + (jnp.arange(NF, dtype=jnp.int32) * V)[None, :]).reshape(-1)
    tables_flat = with_layout_constraint(
        tables.reshape(NF * V, D),
        Layout(major_to_minor=(0, 1), tiling=((8,),)))

    emb = _sc_gather(tables_flat, flat_idx, num_idx).reshape(b, NF * D)

    wpad = jnp.concatenate(
        [w_wide[:N_DENSE], jnp.zeros((NF, 1), jnp.float32), w_wide[N_DENSE:]],
        axis=0)
    bias = (w0 + bo).reshape(1, 1)

    out = pl.pallas_call(
        _mlp_body,
        grid=(b // _BB,),
        in_specs=[
            pl.BlockSpec((_BB, N_IN), lambda i: (i, 0)),
            pl.BlockSpec((_BB, NF * D), lambda i: (i, 0)),
            pl.BlockSpec((NF * D, 256), lambda i: (0, 0)),
            pl.BlockSpec((1, 256), lambda i: (0, 0)),
            pl.BlockSpec((256, 128), lambda i: (0, 0)),
            pl.BlockSpec((1, 128), lambda i: (0, 0)),
            pl.BlockSpec((128, 64), lambda i: (0, 0)),
            pl.BlockSpec((1, 64), lambda i: (0, 0)),
            pl.BlockSpec((64, 1), lambda i: (0, 0)),
            pl.BlockSpec((N_IN, 1), lambda i: (0, 0)),
            pl.BlockSpec((1, 1), lambda i: (0, 0)),
        ],
        out_specs=pl.BlockSpec((_BB, 1), lambda i: (i, 0)),
        out_shape=jax.ShapeDtypeStruct((b, 1), jnp.float32),
    )(inputs, emb, W1, b1.reshape(1, 256), W2, b2.reshape(1, 128), W3,
      b3.reshape(1, 64), Wo, wpad, bias)
    return out
